# baseline (speedup 1.0000x reference)
import functools

import jax
import jax.numpy as jnp
from jax import lax
from jax.experimental import pallas as pl
from jax.experimental.pallas import tpu as pltpu
from jax.experimental.pallas import tpu_sc as plsc

_POOL = 10
_TOPK = 2
_BB = 8
_SC_ROWS_PER_TILE = 2
_CHUNK_T = 16


def _max_body(x_ref, xm_ref):
    xm_ref[...] = jnp.max(x_ref[...], axis=1)


def _tc_max(x_embed, n_rows):
    B, L, D = x_embed.shape
    return pl.pallas_call(
        _max_body,
        grid=(n_rows // _BB,),
        in_specs=[pl.BlockSpec((_BB, L, D), lambda i: (i, 0, 0))],
        out_specs=pl.BlockSpec((_BB, D), lambda i: (i, 0)),
        out_shape=jax.ShapeDtypeStruct((n_rows, D), jnp.float32),
    )(x_embed)


def _make_sc_max(L, D, row0, rows_per_tile, T):
    info = plsc.get_sparse_core_info()
    nw = info.num_cores * info.num_subcores
    lanes = info.num_lanes
    n_rows = nw * rows_per_tile
    ngroups = D // lanes
    npairs = L // T // 2
    mesh = plsc.VectorSubcoreMesh(core_axis_name="c", subcore_axis_name="s")

    @functools.partial(
        pl.kernel, mesh=mesh,
        out_type=jax.ShapeDtypeStruct((n_rows, D), jnp.float32),
        scratch_types=[
            pltpu.VMEM((T, D), jnp.float32),
            pltpu.VMEM((T, D), jnp.float32),
            pltpu.VMEM((1, D), jnp.float32),
            pltpu.SemaphoreType.DMA,
            pltpu.SemaphoreType.DMA,
        ],
    )
    def sc_max(x_hbm, out_hbm, buf0, buf1, acc, sem0, sem1):
        wid = lax.axis_index("s") * info.num_cores + lax.axis_index("c")

        def compute(buf):
            for g in range(ngroups):
                sl = pl.ds(g * lanes, lanes)
                v = [buf[t, sl] for t in range(T)]
                while len(v) > 1:
                    v = [jnp.maximum(v[i], v[i + 1])
                         for i in range(0, len(v), 2)]
                acc[0, sl] = jnp.maximum(acc[0, sl], v[0])

        def row_body(j, carry):
            r_out = wid * rows_per_tile + j
            r_in = row0 + r_out
            for g in range(ngroups):
                acc[0, pl.ds(g * lanes, lanes)] = jnp.full(
                    (lanes,), -jnp.inf, jnp.float32)
            pltpu.async_copy(x_hbm.at[r_in, pl.ds(0, T), :], buf0, sem0)
            pltpu.async_copy(x_hbm.at[r_in, pl.ds(T, T), :], buf1, sem1)

            def pair(p, c):
                base = 2 * p * T
                pltpu.make_async_copy(
                    x_hbm.at[r_in, pl.ds(0, T), :], buf0, sem0).wait()
                compute(buf0)

                @pl.when(p < npairs - 1)
                def _():
                    pltpu.async_copy(
                        x_hbm.at[r_in, pl.ds(base + 2 * T, T), :], buf0, sem0)

                pltpu.make_async_copy(
                    x_hbm.at[r_in, pl.ds(0, T), :], buf1, sem1).wait()
                compute(buf1)

                @pl.when(p < npairs - 1)
                def _():
                    pltpu.async_copy(
                        x_hbm.at[r_in, pl.ds(base + 3 * T, T), :], buf1, sem1)

                return c

            lax.fori_loop(0, npairs, pair, 0)
            pltpu.sync_copy(acc, out_hbm.at[pl.ds(r_out, 1)])
            return carry

        lax.fori_loop(0, rows_per_tile, row_body, 0)

    return sc_max


def _epi_body(xm_ref, pk_ref, p_ref,
              sim_ref, idx_ref, bkn_ref, pkn_ref, xn_ref, rs_ref, bp_ref):
    xm = xm_ref[...]
    nb = xm.shape[0]
    xss = jnp.sum(xm * xm, axis=-1, keepdims=True)
    xn = xm * jax.lax.rsqrt(jnp.maximum(xss, 1e-12))
    pk = pk_ref[...]
    pss = jnp.sum(pk * pk, axis=-1, keepdims=True)
    pkn = pk * jax.lax.rsqrt(jnp.maximum(pss, 1e-12))
    pkn_ref[...] = pkn
    xn_ref[...] = xn
    sim = jax.lax.dot_general(xn, pkn, (((1,), (1,)), ((), ())),
                              preferred_element_type=jnp.float32)
    sim_ref[...] = sim
    cols = jax.lax.broadcasted_iota(jnp.int32, sim.shape, 1)
    v1 = jnp.max(sim, axis=1, keepdims=True)
    i1 = jnp.min(jnp.where(sim == v1, cols, _POOL), axis=1, keepdims=True)
    sim_m = jnp.where(cols == i1, -jnp.inf, sim)
    v2 = jnp.max(sim_m, axis=1, keepdims=True)
    i2 = jnp.min(jnp.where(sim_m == v2, cols, _POOL), axis=1, keepdims=True)
    idx_ref[...] = jnp.concatenate([i1, i2], axis=1)

    p_all = p_ref[...]
    for k, ik in enumerate((i1, i2)):
        gk = jnp.zeros((nb, pkn.shape[1]), jnp.float32)
        gp = jnp.zeros((nb, pkn.shape[1]), jnp.float32)
        for p in range(_POOL):
            m = ik == p
            gk = gk + jnp.where(m, pkn[p:p + 1, :], 0.0)
            gp = gp + jnp.where(m, p_all[p:p + 1, :], 0.0)
        bkn_ref[:, k, :] = gk
        bp_ref[:, k, :] = gp

    rs_ref[...] = jnp.zeros_like(rs_ref) + (jnp.sum(v1) + jnp.sum(v2))


def _epilogue(xm, prompt, prompt_key):
    B, D = xm.shape
    return pl.pallas_call(
        _epi_body,
        grid=(1,),
        in_specs=[
            pl.BlockSpec((B, D), lambda i: (0, 0)),
            pl.BlockSpec((_POOL, D), lambda i: (0, 0)),
            pl.BlockSpec((_POOL, D), lambda i: (0, 0)),
        ],
        out_specs=[
            pl.BlockSpec((B, _POOL), lambda i: (0, 0)),
            pl.BlockSpec((B, _TOPK), lambda i: (0, 0)),
            pl.BlockSpec((B, _TOPK, D), lambda i: (0, 0, 0)),
            pl.BlockSpec((_POOL, D), lambda i: (0, 0)),
            pl.BlockSpec((B, D), lambda i: (0, 0)),
            pl.BlockSpec((1, 1), lambda i: (0, 0)),
            pl.BlockSpec((B, _TOPK, D), lambda i: (0, 0, 0)),
        ],
        out_shape=[
            jax.ShapeDtypeStruct((B, _POOL), jnp.float32),
            jax.ShapeDtypeStruct((B, _TOPK), jnp.int32),
            jax.ShapeDtypeStruct((B, _TOPK, D), jnp.float32),
            jax.ShapeDtypeStruct((_POOL, D), jnp.float32),
            jax.ShapeDtypeStruct((B, D), jnp.float32),
            jax.ShapeDtypeStruct((1, 1), jnp.float32),
            jax.ShapeDtypeStruct((B, _TOPK, D), jnp.float32),
        ],
    )(xm, prompt_key, prompt)


def kernel(x_embed, prompt, prompt_key):
    B, L, D = x_embed.shape
    info = plsc.get_sparse_core_info()
    n_sc = info.num_cores * info.num_subcores * _SC_ROWS_PER_TILE
    n_tc = B - n_sc
    xm_tc = _tc_max(x_embed, n_tc)
    xm_sc = _make_sc_max(L, D, n_tc, _SC_ROWS_PER_TILE, _CHUNK_T)(x_embed)
    xm = jnp.concatenate([xm_tc, xm_sc], axis=0)
    sim, idx, bkn, pkn, xn, rs, bp = _epilogue(xm, prompt, prompt_key)
    reduce_sim = (rs[0, 0] / B).astype(jnp.float32).reshape(())
    return (sim, idx, bkn, pkn, xn, reduce_sim, bp)


# speedup vs baseline: 1.0548x; 1.0548x over previous
import functools

import jax
import jax.numpy as jnp
from jax import lax
from jax.experimental import pallas as pl
from jax.experimental.pallas import tpu as pltpu
from jax.experimental.pallas import tpu_sc as plsc

_POOL = 10
_TOPK = 2
_BB = 8
_SC_ROWS_PER_TILE = 2
_CHUNK_T = 16


def _max_body(x_ref, xm_ref):
    xm_ref[...] = jnp.max(x_ref[...], axis=1)


def _tc_max(x_embed, n_rows):
    B, L, D = x_embed.shape
    return pl.pallas_call(
        _max_body,
        grid=(n_rows // _BB,),
        in_specs=[pl.BlockSpec((_BB, L, D), lambda i: (i, 0, 0))],
        out_specs=pl.BlockSpec((_BB, D), lambda i: (i, 0)),
        out_shape=jax.ShapeDtypeStruct((n_rows, D), jnp.float32),
    )(x_embed)


def _make_sc_max(L, D, row0, rows_per_tile, T):
    info = plsc.get_sparse_core_info()
    nw = info.num_cores * info.num_subcores
    lanes = info.num_lanes
    n_rows = nw * rows_per_tile
    ngroups = D // lanes
    npairs = L // T // 2
    mesh = plsc.VectorSubcoreMesh(core_axis_name="c", subcore_axis_name="s")

    @functools.partial(
        pl.kernel, mesh=mesh,
        out_type=jax.ShapeDtypeStruct((n_rows, D), jnp.float32),
        scratch_types=[
            pltpu.VMEM((T, D), jnp.float32),
            pltpu.VMEM((T, D), jnp.float32),
            pltpu.VMEM((1, D), jnp.float32),
            pltpu.SemaphoreType.DMA,
            pltpu.SemaphoreType.DMA,
        ],
    )
    def sc_max(x_hbm, out_hbm, buf0, buf1, acc, sem0, sem1):
        wid = lax.axis_index("s") * info.num_cores + lax.axis_index("c")

        def compute(buf):
            for g in range(0, ngroups, 4):
                sls = [pl.ds((g + u) * lanes, lanes) for u in range(4)]
                ms = [acc[0, s] for s in sls]
                for t in range(T):
                    ms = [jnp.maximum(ms[u], buf[t, sls[u]]) for u in range(4)]
                for u in range(4):
                    acc[0, sls[u]] = ms[u]

        def row_body(j, carry):
            r_out = wid * rows_per_tile + j
            r_in = row0 + r_out
            for g in range(ngroups):
                acc[0, pl.ds(g * lanes, lanes)] = jnp.full(
                    (lanes,), -jnp.inf, jnp.float32)
            pltpu.async_copy(x_hbm.at[r_in, pl.ds(0, T), :], buf0, sem0)
            pltpu.async_copy(x_hbm.at[r_in, pl.ds(T, T), :], buf1, sem1)

            def pair(p, c):
                base = 2 * p * T
                pltpu.make_async_copy(
                    x_hbm.at[r_in, pl.ds(0, T), :], buf0, sem0).wait()
                compute(buf0)

                @pl.when(p < npairs - 1)
                def _():
                    pltpu.async_copy(
                        x_hbm.at[r_in, pl.ds(base + 2 * T, T), :], buf0, sem0)

                pltpu.make_async_copy(
                    x_hbm.at[r_in, pl.ds(0, T), :], buf1, sem1).wait()
                compute(buf1)

                @pl.when(p < npairs - 1)
                def _():
                    pltpu.async_copy(
                        x_hbm.at[r_in, pl.ds(base + 3 * T, T), :], buf1, sem1)

                return c

            lax.fori_loop(0, npairs, pair, 0)
            pltpu.sync_copy(acc, out_hbm.at[pl.ds(r_out, 1)])
            return carry

        lax.fori_loop(0, rows_per_tile, row_body, 0)

    return sc_max


def _epi_body(xm_ref, pk_ref, p_ref,
              sim_ref, idx_ref, bkn_ref, pkn_ref, xn_ref, rs_ref, bp_ref):
    xm = xm_ref[...]
    nb = xm.shape[0]
    xss = jnp.sum(xm * xm, axis=-1, keepdims=True)
    xn = xm * jax.lax.rsqrt(jnp.maximum(xss, 1e-12))
    pk = pk_ref[...]
    pss = jnp.sum(pk * pk, axis=-1, keepdims=True)
    pkn = pk * jax.lax.rsqrt(jnp.maximum(pss, 1e-12))
    pkn_ref[...] = pkn
    xn_ref[...] = xn
    sim = jax.lax.dot_general(xn, pkn, (((1,), (1,)), ((), ())),
                              preferred_element_type=jnp.float32)
    sim_ref[...] = sim
    cols = jax.lax.broadcasted_iota(jnp.int32, sim.shape, 1)
    v1 = jnp.max(sim, axis=1, keepdims=True)
    i1 = jnp.min(jnp.where(sim == v1, cols, _POOL), axis=1, keepdims=True)
    sim_m = jnp.where(cols == i1, -jnp.inf, sim)
    v2 = jnp.max(sim_m, axis=1, keepdims=True)
    i2 = jnp.min(jnp.where(sim_m == v2, cols, _POOL), axis=1, keepdims=True)
    idx_ref[...] = jnp.concatenate([i1, i2], axis=1)

    p_all = p_ref[...]
    for k, ik in enumerate((i1, i2)):
        gk = jnp.zeros((nb, pkn.shape[1]), jnp.float32)
        gp = jnp.zeros((nb, pkn.shape[1]), jnp.float32)
        for p in range(_POOL):
            m = ik == p
            gk = gk + jnp.where(m, pkn[p:p + 1, :], 0.0)
            gp = gp + jnp.where(m, p_all[p:p + 1, :], 0.0)
        bkn_ref[:, k, :] = gk
        bp_ref[:, k, :] = gp

    rs_ref[...] = jnp.zeros_like(rs_ref) + (jnp.sum(v1) + jnp.sum(v2))


def _epilogue(xm, prompt, prompt_key):
    B, D = xm.shape
    return pl.pallas_call(
        _epi_body,
        grid=(1,),
        in_specs=[
            pl.BlockSpec((B, D), lambda i: (0, 0)),
            pl.BlockSpec((_POOL, D), lambda i: (0, 0)),
            pl.BlockSpec((_POOL, D), lambda i: (0, 0)),
        ],
        out_specs=[
            pl.BlockSpec((B, _POOL), lambda i: (0, 0)),
            pl.BlockSpec((B, _TOPK), lambda i: (0, 0)),
            pl.BlockSpec((B, _TOPK, D), lambda i: (0, 0, 0)),
            pl.BlockSpec((_POOL, D), lambda i: (0, 0)),
            pl.BlockSpec((B, D), lambda i: (0, 0)),
            pl.BlockSpec((1, 1), lambda i: (0, 0)),
            pl.BlockSpec((B, _TOPK, D), lambda i: (0, 0, 0)),
        ],
        out_shape=[
            jax.ShapeDtypeStruct((B, _POOL), jnp.float32),
            jax.ShapeDtypeStruct((B, _TOPK), jnp.int32),
            jax.ShapeDtypeStruct((B, _TOPK, D), jnp.float32),
            jax.ShapeDtypeStruct((_POOL, D), jnp.float32),
            jax.ShapeDtypeStruct((B, D), jnp.float32),
            jax.ShapeDtypeStruct((1, 1), jnp.float32),
            jax.ShapeDtypeStruct((B, _TOPK, D), jnp.float32),
        ],
    )(xm, prompt_key, prompt)


def kernel(x_embed, prompt, prompt_key):
    B, L, D = x_embed.shape
    info = plsc.get_sparse_core_info()
    n_sc = info.num_cores * info.num_subcores * _SC_ROWS_PER_TILE
    n_tc = B - n_sc
    xm_tc = _tc_max(x_embed, n_tc)
    xm_sc = _make_sc_max(L, D, n_tc, _SC_ROWS_PER_TILE, _CHUNK_T)(x_embed)
    xm = jnp.concatenate([xm_tc, xm_sc], axis=0)
    sim, idx, bkn, pkn, xn, rs, bp = _epilogue(xm, prompt, prompt_key)
    reduce_sim = (rs[0, 0] / B).astype(jnp.float32).reshape(())
    return (sim, idx, bkn, pkn, xn, reduce_sim, bp)


# speedup vs baseline: 1.6459x; 1.5605x over previous
import jax
import jax.numpy as jnp
from jax.experimental import pallas as pl

_POOL = 10
_TOPK = 2
_BB = 8


def _eprompt_body(x_ref, pk_ref, p_ref,
                  sim_ref, idx_ref, bkn_ref, pkn_ref, xn_ref, rs_ref, bp_ref):
    xm = jnp.max(x_ref[...], axis=1)
    xss = jnp.sum(xm * xm, axis=-1, keepdims=True)
    xn = xm * jax.lax.rsqrt(jnp.maximum(xss, 1e-12))
    pk = pk_ref[...]
    pss = jnp.sum(pk * pk, axis=-1, keepdims=True)
    pkn = pk * jax.lax.rsqrt(jnp.maximum(pss, 1e-12))
    pkn_ref[...] = pkn
    xn_ref[...] = xn
    sim = jax.lax.dot_general(xn, pkn, (((1,), (1,)), ((), ())),
                              preferred_element_type=jnp.float32)
    sim_ref[...] = sim
    cols = jax.lax.broadcasted_iota(jnp.int32, sim.shape, 1)
    v1 = jnp.max(sim, axis=1, keepdims=True)
    i1 = jnp.min(jnp.where(sim == v1, cols, _POOL), axis=1, keepdims=True)
    sim_m = jnp.where(cols == i1, -jnp.inf, sim)
    v2 = jnp.max(sim_m, axis=1, keepdims=True)
    i2 = jnp.min(jnp.where(sim_m == v2, cols, _POOL), axis=1, keepdims=True)
    idx_ref[...] = jnp.concatenate([i1, i2], axis=1)

    p_all = p_ref[...]
    for k, ik in enumerate((i1, i2)):
        gk = jnp.zeros((_BB, pkn.shape[1]), jnp.float32)
        gp = jnp.zeros((_BB, pkn.shape[1]), jnp.float32)
        for p in range(_POOL):
            m = ik == p
            gk = gk + jnp.where(m, pkn[p:p + 1, :], 0.0)
            gp = gp + jnp.where(m, p_all[p:p + 1, :], 0.0)
        bkn_ref[:, k, :] = gk
        bp_ref[:, k, :] = gp

    @pl.when(pl.program_id(0) == 0)
    def _():
        rs_ref[...] = jnp.zeros_like(rs_ref)

    rs_ref[...] = rs_ref[...] + (jnp.sum(v1) + jnp.sum(v2))


def kernel(x_embed, prompt, prompt_key):
    B, L, D = x_embed.shape
    grid = (B // _BB,)
    outs = pl.pallas_call(
        _eprompt_body,
        grid=grid,
        in_specs=[
            pl.BlockSpec((_BB, L, D), lambda i: (i, 0, 0)),
            pl.BlockSpec((_POOL, D), lambda i: (0, 0)),
            pl.BlockSpec((_POOL, D), lambda i: (0, 0)),
        ],
        out_specs=[
            pl.BlockSpec((_BB, _POOL), lambda i: (i, 0)),
            pl.BlockSpec((_BB, _TOPK), lambda i: (i, 0)),
            pl.BlockSpec((_BB, _TOPK, D), lambda i: (i, 0, 0)),
            pl.BlockSpec((_POOL, D), lambda i: (0, 0)),
            pl.BlockSpec((_BB, D), lambda i: (i, 0)),
            pl.BlockSpec((1, 1), lambda i: (0, 0)),
            pl.BlockSpec((_BB, _TOPK, D), lambda i: (i, 0, 0)),
        ],
        out_shape=[
            jax.ShapeDtypeStruct((B, _POOL), jnp.float32),
            jax.ShapeDtypeStruct((B, _TOPK), jnp.int32),
            jax.ShapeDtypeStruct((B, _TOPK, D), jnp.float32),
            jax.ShapeDtypeStruct((_POOL, D), jnp.float32),
            jax.ShapeDtypeStruct((B, D), jnp.float32),
            jax.ShapeDtypeStruct((1, 1), jnp.float32),
            jax.ShapeDtypeStruct((B, _TOPK, D), jnp.float32),
        ],
    )(x_embed, prompt_key, prompt)
    sim, idx, bkn, pkn, xn, rs, bp = outs
    reduce_sim = (rs[0, 0] / B).astype(jnp.float32).reshape(())
    return (sim, idx, bkn, pkn, xn, reduce_sim, bp)
